# baseline (speedup 1.0000x reference)
import functools

import jax
import jax.numpy as jnp
from jax import lax
from jax.experimental import pallas as pl
from jax.experimental.pallas import tpu as pltpu
from jax.experimental.pallas import tpu_sc as plsc

B = 4096
L = 50
D = 64
NW = 32
BPW = B // NW
NJ = D // 16
NBUF = 2

_mesh = plsc.VectorSubcoreMesh(core_axis_name="c", subcore_axis_name="s")


@functools.partial(
    pl.kernel,
    mesh=_mesh,
    compiler_params=pltpu.CompilerParams(use_tc_tiling_on_sc=False),
    out_type=jax.ShapeDtypeStruct((B, D), jnp.float32),
    scratch_types=[
        pltpu.VMEM((BPW, L), jnp.int32),
        pltpu.VMEM((NBUF, L, D), jnp.float32),
        pltpu.VMEM((BPW, D), jnp.float32),
        pltpu.SemaphoreType.DMA,
        pltpu.SemaphoreType.DMA,
    ],
)
def _mean_embed(table_hbm, x_hbm, out_hbm, idx_v, rows_v, out_v, sem0, sem1):
    wid = lax.axis_index("s") * 2 + lax.axis_index("c")
    pltpu.sync_copy(x_hbm.at[pl.ds(wid * BPW, BPW)], idx_v)
    sems = (sem0, sem1)

    def gather(c, b):
        return pltpu.make_async_copy(
            table_hbm.at[idx_v.at[c]], rows_v.at[b], sems[b])

    gather(0, 0).start()
    gather(1, 1).start()

    def step(i, carry):
        for b in range(NBUF):
            c = NBUF * i + b
            gather(c, b).wait()
            acc = [None] * (2 * NJ)
            for l in range(L):
                for j in range(NJ):
                    v = rows_v[b, l, pl.ds(j * 16, 16)]
                    k = (l % 2) * NJ + j
                    acc[k] = v if acc[k] is None else acc[k] + v
            for j in range(NJ):
                out_v[c, pl.ds(j * 16, 16)] = (acc[j] + acc[NJ + j]) * (1.0 / L)

            @pl.when(c + NBUF < BPW)
            def _():
                gather(c + NBUF, b).start()
        return carry

    lax.fori_loop(0, BPW // NBUF, step, 0)
    pltpu.sync_copy(out_v, out_hbm.at[pl.ds(wid * BPW, BPW)])


def kernel(vectors, x):
    return _mean_embed(vectors, x.astype(jnp.int32))


# speedup vs baseline: 1.1393x; 1.1393x over previous
import functools

import jax
import jax.numpy as jnp
from jax import lax
from jax.experimental import pallas as pl
from jax.experimental.pallas import tpu as pltpu
from jax.experimental.pallas import tpu_sc as plsc

B = 4096
L = 50
D = 64
NW = 32
BPW = B // NW
IPW = BPW * L
NJ = D // 16
NBUF = 2
RUN = 8

_mesh = plsc.VectorSubcoreMesh(core_axis_name="c", subcore_axis_name="s")


@functools.partial(
    pl.kernel,
    mesh=_mesh,
    compiler_params=pltpu.CompilerParams(use_tc_tiling_on_sc=False),
    out_type=jax.ShapeDtypeStruct((B, D), jnp.float32),
    scratch_types=[
        pltpu.VMEM((IPW,), jnp.int32),
        pltpu.VMEM((NBUF, BPW, D), jnp.float32),
        pltpu.VMEM((BPW, D), jnp.float32),
        pltpu.SemaphoreType.DMA,
        pltpu.SemaphoreType.DMA,
    ],
)
def _mean_embed(table_hbm, xt_hbm, out_hbm, idx_v, rows_v, acc_v, sem0, sem1):
    wid = lax.axis_index("s") * 2 + lax.axis_index("c")
    pltpu.sync_copy(xt_hbm.at[pl.ds(wid * IPW, IPW)], idx_v)
    sems = (sem0, sem1)

    def gather(l, b):
        return pltpu.make_async_copy(
            table_hbm.at[idx_v.at[pl.ds(l * BPW, BPW)]], rows_v.at[b], sems[b])

    zero = jnp.zeros((16,), jnp.float32)

    def zero_block(g, carry):
        for rr in range(RUN):
            for j in range(NJ):
                acc_v[g * RUN + rr, pl.ds(j * 16, 16)] = zero
        return carry

    lax.fori_loop(0, BPW // RUN, zero_block, 0)

    gather(0, 0).start()
    gather(1, 1).start()

    def accum(l, b):
        gather(l, b).wait()

        def row_block(g, carry):
            for rr in range(RUN):
                r = g * RUN + rr
                for j in range(NJ):
                    plsc.addupdate(
                        acc_v.at[r, pl.ds(j * 16, 16)],
                        rows_v[b, r, pl.ds(j * 16, 16)])
            return carry

        lax.fori_loop(0, BPW // RUN, row_block, 0)

        @pl.when(l + NBUF < L)
        def _():
            gather(l + NBUF, b).start()

    def step(i, carry):
        for b in range(NBUF):
            accum(NBUF * i + b, b)
        return carry

    lax.fori_loop(0, L // NBUF, step, 0)

    def scale_block(g, carry):
        for rr in range(RUN):
            r = g * RUN + rr
            for j in range(NJ):
                acc_v[r, pl.ds(j * 16, 16)] = (
                    acc_v[r, pl.ds(j * 16, 16)] * (1.0 / L))
        return carry

    lax.fori_loop(0, BPW // RUN, scale_block, 0)
    pltpu.sync_copy(acc_v, out_hbm.at[pl.ds(wid * BPW, BPW)])


def kernel(vectors, x):
    xt = (x.astype(jnp.int32)
           .reshape(NW, BPW, L)
           .transpose(0, 2, 1)
           .reshape(NW * IPW))
    return _mean_embed(vectors, xt)


# speedup vs baseline: 1.1667x; 1.0241x over previous
import functools

import jax
import jax.numpy as jnp
from jax import lax
from jax.experimental import pallas as pl
from jax.experimental.pallas import tpu as pltpu
from jax.experimental.pallas import tpu_sc as plsc

B = 4096
L = 50
D = 64
NW = 32
NS = 16
BPW = B // NW
IPW = BPW * L
CH = 128
NCH = IPW // CH
NJ = D // 16
NBUF = 5
RUN = 8

_mesh = plsc.VectorSubcoreMesh(core_axis_name="c", subcore_axis_name="s")


@functools.partial(
    pl.kernel,
    mesh=_mesh,
    compiler_params=pltpu.CompilerParams(use_tc_tiling_on_sc=False),
    out_type=jax.ShapeDtypeStruct((B, D), jnp.float32),
    scratch_types=[
        pltpu.VMEM((IPW,), jnp.int32),
        pltpu.VMEM((IPW,), jnp.int32),
        pltpu.VMEM((NBUF, CH, D), jnp.float32),
        pltpu.VMEM((BPW, D), jnp.float32),
        pltpu.VMEM_SHARED((NS * BPW, D), jnp.float32),
        pltpu.SemaphoreType.DMA,
        pltpu.SemaphoreType.DMA,
        pltpu.SemaphoreType.DMA,
        pltpu.SemaphoreType.DMA,
        pltpu.SemaphoreType.DMA,
        pltpu.SemaphoreType.DMA,
    ],
)
def _mean_embed(table_hbm, x_hbm, d_hbm, out_hbm,
                idx_v, d_v, rows_v, stage_v, acc_sh,
                gs0, gs1, gs2, gs3, gs4, ssem):
    cid = lax.axis_index("c")
    sid = lax.axis_index("s")
    wid = sid * 2 + cid
    pltpu.sync_copy(x_hbm.at[pl.ds(wid * IPW, IPW)], idx_v)
    pltpu.sync_copy(d_hbm, d_v)
    gsems = (gs0, gs1, gs2, gs3, gs4)

    base = (sid * BPW).astype(jnp.int32)
    zero = jnp.zeros((16,), jnp.float32)

    def prep_block(g, carry):
        for rr in range(RUN):
            r = g * RUN + rr
            for j in range(NJ):
                stage_v[r, pl.ds(j * 16, 16)] = zero
        for j in range(RUN * L // 16):
            q = g * (RUN * L // 16) + j
            d_v[pl.ds(q * 16, 16)] = d_v[pl.ds(q * 16, 16)] + base
        return carry

    lax.fori_loop(0, BPW // RUN, prep_block, 0)
    pltpu.sync_copy(stage_v, acc_sh.at[pl.ds(sid * BPW, BPW)])

    def gather(c, b):
        return pltpu.make_async_copy(
            table_hbm.at[idx_v.at[pl.ds(c * CH, CH)]], rows_v.at[b], gsems[b])

    for b in range(NBUF):
        gather(b, b).start()

    def step(i, carry):
        for b in range(NBUF):
            c = i * NBUF + b
            gather(c, b).wait()
            pltpu.async_copy(
                rows_v.at[b], acc_sh.at[d_v.at[pl.ds(c * CH, CH)]], ssem,
                add=True).wait()

            @pl.when(c + NBUF < NCH)
            def _():
                gather(c + NBUF, b).start()

        return carry

    lax.fori_loop(0, NCH // NBUF, step, 0)
    pltpu.sync_copy(acc_sh.at[pl.ds(sid * BPW, BPW)], stage_v)

    def scale_block(g, carry):
        for rr in range(RUN):
            r = g * RUN + rr
            for j in range(NJ):
                stage_v[r, pl.ds(j * 16, 16)] = (
                    stage_v[r, pl.ds(j * 16, 16)] * (1.0 / L))
        return carry

    lax.fori_loop(0, BPW // RUN, scale_block, 0)
    pltpu.sync_copy(stage_v, out_hbm.at[pl.ds(wid * BPW, BPW)])


def kernel(vectors, x):
    xf = x.astype(jnp.int32).reshape(B * L)
    d = (jnp.arange(IPW, dtype=jnp.int32) // L).astype(jnp.int32)
    return _mean_embed(vectors, xf, d)
